# Initial kernel scaffold; baseline (speedup 1.0000x reference)
import jax
import jax.numpy as jnp
from jax.experimental import pallas as pl


def kernel(x, edge_index, W1, b1, W2, b2, gamma):
    raise NotImplementedError("write your pallas kernel here")


# speedup vs baseline: 6.5826x; 6.5826x over previous
import functools

import jax
import jax.numpy as jnp
from jax import lax
from jax.experimental import pallas as pl
from jax.experimental.pallas import tpu as pltpu
from jax.experimental.pallas import tpu_sc as plsc

NC = 2
NS = 16
LANES = 16
ECHUNK = 128


def _round_up(a, b):
    return ((a + b - 1) // b) * b


def _sc_degree(dst_r, NP, CHUNKS):
    mesh = plsc.VectorSubcoreMesh(core_axis_name="c", subcore_axis_name="s")
    rpt = NP // NS

    @functools.partial(
        pl.kernel,
        out_type=jax.ShapeDtypeStruct((NC, NP), jnp.float32),
        mesh=mesh,
        scratch_types=dict(
            deg_sp=pltpu.VMEM_SHARED((NP,), jnp.float32),
            didx=pltpu.VMEM((1, ECHUNK), jnp.int32),
            ones_v=pltpu.VMEM((ECHUNK,), jnp.float32),
            zbuf=pltpu.VMEM((rpt,), jnp.float32),
        ),
    )
    def deg_kernel(dst_hbm, out_hbm, deg_sp, didx, ones_v, zbuf):
        cid = lax.axis_index("c")
        sid = lax.axis_index("s")

        def _fill(i, _):
            zbuf[pl.ds(i * LANES, LANES)] = jnp.zeros((LANES,), jnp.float32)
            return _

        lax.fori_loop(0, rpt // LANES, _fill, None)

        def _fill_ones(i, _):
            ones_v[pl.ds(i * LANES, LANES)] = jnp.ones((LANES,), jnp.float32)
            return _

        lax.fori_loop(0, ECHUNK // LANES, _fill_ones, None)
        pltpu.sync_copy(zbuf, deg_sp.at[pl.ds(sid * rpt, rpt)])
        plsc.subcore_barrier()

        def _edge(c, _):
            pltpu.sync_copy(dst_hbm.at[cid, sid, c], didx.at[0])
            pltpu.sync_copy(ones_v, deg_sp.at[didx.at[0]], add=True)
            return _

        lax.fori_loop(0, CHUNKS, _edge, None)
        plsc.subcore_barrier()
        pltpu.sync_copy(deg_sp.at[pl.ds(sid * rpt, rpt)],
                        out_hbm.at[cid, pl.ds(sid * rpt, rpt)])

    return deg_kernel(dst_r)


def _sc_scatter(g, src_r, dst_r, NP, CHUNKS):
    D = g.shape[1]
    mesh = plsc.VectorSubcoreMesh(core_axis_name="c", subcore_axis_name="s")
    rpt = NP // NS
    ZR = 64

    @functools.partial(
        pl.kernel,
        out_type=jax.ShapeDtypeStruct((NC, NP, D), jnp.float32),
        mesh=mesh,
        scratch_types=dict(
            acc_sp=pltpu.VMEM_SHARED((NP, D), jnp.float32),
            sidx=pltpu.VMEM((1, ECHUNK), jnp.int32),
            didx=pltpu.VMEM((1, ECHUNK), jnp.int32),
            rows=pltpu.VMEM((ECHUNK, D), jnp.float32),
            zbuf=pltpu.VMEM((ZR, D), jnp.float32),
            sem=pltpu.SemaphoreType.DMA,
        ),
    )
    def round_kernel(g_hbm, src_hbm, dst_hbm, out_hbm,
                     acc_sp, sidx, didx, rows, zbuf, sem):
        cid = lax.axis_index("c")
        sid = lax.axis_index("s")

        def _fill(i, _):
            r = i // (D // LANES)
            j = i % (D // LANES)
            zbuf[r, pl.ds(j * LANES, LANES)] = jnp.zeros((LANES,), jnp.float32)
            return _

        lax.fori_loop(0, ZR * (D // LANES), _fill, None)

        def _zero(t, _):
            pltpu.sync_copy(zbuf, acc_sp.at[pl.ds(sid * rpt + t * ZR, ZR)])
            return _

        lax.fori_loop(0, rpt // ZR, _zero, None)
        plsc.subcore_barrier()

        def _edge(c, _):
            pltpu.sync_copy(src_hbm.at[cid, sid, c], sidx.at[0])
            pltpu.sync_copy(dst_hbm.at[cid, sid, c], didx.at[0])
            pltpu.async_copy(g_hbm.at[sidx.at[0]], rows, sem).wait()
            pltpu.sync_copy(rows, acc_sp.at[didx.at[0]], add=True)
            return _

        lax.fori_loop(0, CHUNKS, _edge, None)
        plsc.subcore_barrier()
        pltpu.sync_copy(acc_sp.at[pl.ds(sid * rpt, rpt)],
                        out_hbm.at[cid, pl.ds(sid * rpt, rpt)])

    return round_kernel(g, src_r, dst_r)


def _tc_mlp(xp, W1, b1r, W2, b2r, degp, N, BM=512):
    NP, D = xp.shape

    def body(x_ref, w1_ref, b1_ref, w2_ref, b2_ref, degp_ref,
             h0_ref, g0_ref, dinv_ref, s_ref):
        i = pl.program_id(0)
        h = lax.dot_general(x_ref[...], w1_ref[...], (((1,), (1,)), ((), ())),
                            precision=lax.Precision.HIGHEST)
        h = jnp.maximum(h + b1_ref[...], 0.0)
        h = lax.dot_general(h, w2_ref[...], (((1,), (1,)), ((), ())),
                            precision=lax.Precision.HIGHEST)
        h = h + b2_ref[...]
        rows = i * BM + lax.broadcasted_iota(jnp.int32, (BM, 1), 0)
        h = jnp.where(rows < N, h, 0.0)
        deg = 1.0 + jnp.sum(degp_ref[...], axis=0, keepdims=True)
        s = lax.rsqrt(deg)
        dinv = 1.0 / deg
        h0_ref[...] = h
        g0_ref[...] = h * s.T
        dinv_ref[...] = dinv.T
        s_ref[...] = s.T

    grid = (NP // BM,)
    return pl.pallas_call(
        body,
        grid=grid,
        in_specs=[
            pl.BlockSpec((BM, D), lambda i: (i, 0)),
            pl.BlockSpec((D, D), lambda i: (0, 0)),
            pl.BlockSpec((1, D), lambda i: (0, 0)),
            pl.BlockSpec((D, D), lambda i: (0, 0)),
            pl.BlockSpec((1, D), lambda i: (0, 0)),
            pl.BlockSpec((NC, BM), lambda i: (0, i)),
        ],
        out_specs=[
            pl.BlockSpec((BM, D), lambda i: (i, 0)),
            pl.BlockSpec((BM, D), lambda i: (i, 0)),
            pl.BlockSpec((BM, 1), lambda i: (i, 0)),
            pl.BlockSpec((BM, 1), lambda i: (i, 0)),
        ],
        out_shape=[
            jax.ShapeDtypeStruct((NP, D), jnp.float32),
            jax.ShapeDtypeStruct((NP, D), jnp.float32),
            jax.ShapeDtypeStruct((NP, 1), jnp.float32),
            jax.ShapeDtypeStruct((NP, 1), jnp.float32),
        ],
    )(xp, W1, b1r, W2, b2r, degp)


def _tc_combine(tmpP, g, dinv, acc, gk, BM=512):
    NP, D = g.shape

    def body(tmp_ref, g_ref, dinv_ref, acc_ref, gk_ref, gn_ref, an_ref):
        tmp = tmp_ref[0] + tmp_ref[1] + g_ref[...]
        gn_ref[...] = dinv_ref[...] * tmp
        an_ref[...] = acc_ref[...] + gk_ref[0, 0] * tmp

    return pl.pallas_call(
        body,
        grid=(NP // BM,),
        in_specs=[
            pl.BlockSpec((NC, BM, D), lambda i: (0, i, 0)),
            pl.BlockSpec((BM, D), lambda i: (i, 0)),
            pl.BlockSpec((BM, 1), lambda i: (i, 0)),
            pl.BlockSpec((BM, D), lambda i: (i, 0)),
            pl.BlockSpec((1, 1), lambda i: (0, 0), memory_space=pltpu.SMEM),
        ],
        out_specs=[
            pl.BlockSpec((BM, D), lambda i: (i, 0)),
            pl.BlockSpec((BM, D), lambda i: (i, 0)),
        ],
        out_shape=[
            jax.ShapeDtypeStruct((NP, D), jnp.float32),
            jax.ShapeDtypeStruct((NP, D), jnp.float32),
        ],
    )(tmpP, g, dinv, acc, gk)


def _tc_final(tmpP, g, acc, s, h0, gk, g0scal, BM=512):
    NP, D = g.shape

    def body(tmp_ref, g_ref, acc_ref, s_ref, h0_ref, gk_ref, g0_ref, out_ref):
        tmp = tmp_ref[0] + tmp_ref[1] + g_ref[...]
        acc = acc_ref[...] + gk_ref[0, 0] * tmp
        out_ref[...] = g0_ref[0, 0] * h0_ref[...] + s_ref[...] * acc

    return pl.pallas_call(
        body,
        grid=(NP // BM,),
        in_specs=[
            pl.BlockSpec((NC, BM, D), lambda i: (0, i, 0)),
            pl.BlockSpec((BM, D), lambda i: (i, 0)),
            pl.BlockSpec((BM, D), lambda i: (i, 0)),
            pl.BlockSpec((BM, 1), lambda i: (i, 0)),
            pl.BlockSpec((BM, D), lambda i: (i, 0)),
            pl.BlockSpec((1, 1), lambda i: (0, 0), memory_space=pltpu.SMEM),
            pl.BlockSpec((1, 1), lambda i: (0, 0), memory_space=pltpu.SMEM),
        ],
        out_specs=pl.BlockSpec((BM, D), lambda i: (i, 0)),
        out_shape=jax.ShapeDtypeStruct((NP, D), jnp.float32),
    )(tmpP, g, acc, s, h0, gk, g0scal)


def kernel(x, edge_index, W1, b1, W2, b2, gamma):
    N, D = x.shape
    E = edge_index.shape[1]
    K = gamma.shape[0] - 1

    NP = _round_up(N + 1, NS * 64)
    EP = _round_up(E, NC * NS * ECHUNK)
    CHUNKS = EP // (NC * NS * ECHUNK)

    src = jnp.concatenate(
        [edge_index[0], jnp.full((EP - E,), N, jnp.int32)]
    ).reshape(NC, NS, CHUNKS, ECHUNK)
    dst = jnp.concatenate(
        [edge_index[1], jnp.full((EP - E,), NP - 1, jnp.int32)]
    ).reshape(NC, NS, CHUNKS, ECHUNK)
    xp = jnp.pad(x, ((0, NP - N), (0, 0)))

    degp = _sc_degree(dst, NP, CHUNKS)
    h0, g0, dinv, s = _tc_mlp(xp, W1, b1.reshape(1, D), W2,
                              b2.reshape(1, D), degp, N)

    g = g0
    acc = jnp.zeros((NP, D), jnp.float32)
    for k in range(1, K + 1):
        tmpP = _sc_scatter(g, src, dst, NP, CHUNKS)
        gk = gamma[k].reshape(1, 1)
        if k < K:
            g, acc = _tc_combine(tmpP, g, dinv, acc, gk)
        else:
            out = _tc_final(tmpP, g, acc, s, h0, gk, gamma[0].reshape(1, 1))
    return out[:N]


# speedup vs baseline: 11.0943x; 1.6854x over previous
import functools

import jax
import jax.numpy as jnp
from jax import lax
from jax.experimental import pallas as pl
from jax.experimental.pallas import tpu as pltpu
from jax.experimental.pallas import tpu_sc as plsc

NC = 2
NS = 16
LANES = 16
ECHUNK = 112
NBUF = 3
ZCH = 64


def _round_up(a, b):
    return ((a + b - 1) // b) * b


def _sc_degree(ed_r, NP, CHUNKS):
    mesh = plsc.VectorSubcoreMesh(core_axis_name="c", subcore_axis_name="s")
    rpt = NP // NS

    @functools.partial(
        pl.kernel,
        out_type=jax.ShapeDtypeStruct((NC, NP), jnp.float32),
        mesh=mesh,
        scratch_types=dict(
            deg_sp=pltpu.VMEM_SHARED((NP,), jnp.float32),
            ones_v=pltpu.VMEM((ECHUNK,), jnp.float32),
            zbuf=pltpu.VMEM((rpt,), jnp.float32),
        ),
    )
    def deg_kernel(ed_hbm, out_hbm, deg_sp, ones_v, zbuf):
        def scoped(didx):
            cid = lax.axis_index("c")
            sid = lax.axis_index("s")
            pltpu.sync_copy(ed_hbm.at[cid, sid], didx)

            def _fill(i, _):
                zbuf[pl.ds(i * LANES, LANES)] = jnp.zeros((LANES,),
                                                          jnp.float32)
                return _

            lax.fori_loop(0, rpt // LANES, _fill, None)

            def _fill_ones(i, _):
                ones_v[pl.ds(i * LANES, LANES)] = jnp.ones((LANES,),
                                                           jnp.float32)
                return _

            lax.fori_loop(0, ECHUNK // LANES, _fill_ones, None)
            pltpu.sync_copy(zbuf, deg_sp.at[pl.ds(sid * rpt, rpt)])
            plsc.subcore_barrier()

            def _edge(c, _):
                pltpu.sync_copy(ones_v, deg_sp.at[didx.at[c, 1]], add=True)
                return _

            lax.fori_loop(0, CHUNKS, _edge, None)
            plsc.subcore_barrier()
            pltpu.sync_copy(deg_sp.at[pl.ds(sid * rpt, rpt)],
                            out_hbm.at[cid, pl.ds(sid * rpt, rpt)])

        pl.run_scoped(scoped, pltpu.VMEM((CHUNKS, 2, ECHUNK), jnp.int32))

    return deg_kernel(ed_r)


def _sc_scatter(g, ed_r, NP, CHUNKS):
    D = g.shape[1]
    mesh = plsc.VectorSubcoreMesh(core_axis_name="c", subcore_axis_name="s")
    rpt = NP // NS
    NI = 2 * NBUF
    TT = CHUNKS // NI

    @functools.partial(
        pl.kernel,
        out_type=jax.ShapeDtypeStruct((NC, NP, D), jnp.float32),
        mesh=mesh,
        scratch_types=dict(
            acc_sp=pltpu.VMEM_SHARED((NP, D), jnp.float32),
            gsem=pltpu.SemaphoreType.DMA((NBUF,)),
            isem=pltpu.SemaphoreType.DMA((NI,)),
        ),
    )
    def round_kernel(g_hbm, ed_hbm, out_hbm, acc_sp, gsem, isem):
        def scoped(ibuf, *rows):
            _body(g_hbm, ed_hbm, out_hbm, acc_sp, gsem, isem, ibuf, rows)

        pl.run_scoped(
            scoped,
            pltpu.VMEM((NI, 2, ECHUNK), jnp.int32),
            *[pltpu.VMEM((ECHUNK, D), jnp.float32) for _ in range(NBUF)],
        )

    def _body(g_hbm, ed_hbm, out_hbm, acc_sp, gsem, isem, ibuf, rows):
        cid = lax.axis_index("c")
        sid = lax.axis_index("s")

        def _drain_i(s):
            pltpu.make_async_copy(ed_hbm.at[cid, sid, 0], ibuf.at[s],
                                  isem.at[s]).wait()

        def _drain_g(b):
            pltpu.make_async_copy(g_hbm.at[pl.ds(0, ECHUNK)], rows[b],
                                  gsem.at[b]).wait()

        def _fill(i, _):
            r = i // (D // LANES)
            j = i % (D // LANES)
            rows[0][r, pl.ds(j * LANES, LANES)] = jnp.zeros((LANES,),
                                                            jnp.float32)
            return _

        lax.fori_loop(0, ZCH * (D // LANES), _fill, None)

        def _zero(t, _):
            pltpu.sync_copy(rows[0].at[pl.ds(0, ZCH)],
                            acc_sp.at[pl.ds(sid * rpt + t * ZCH, ZCH)])
            return _

        lax.fori_loop(0, rpt // ZCH, _zero, None)
        plsc.subcore_barrier()

        for s in range(NI):
            pltpu.async_copy(ed_hbm.at[cid, sid, s], ibuf.at[s], isem.at[s])
        for b in range(NBUF):
            _drain_i(b)
            pltpu.async_copy(g_hbm.at[ibuf.at[b, 0]], rows[b], gsem.at[b])

        def _visit(tt, _):
            for half in range(2):
                for b in range(NBUF):
                    s = half * NBUF + b
                    s2 = (s + NBUF) % NI
                    c = (tt * 2 + half) * NBUF + b
                    _drain_g(b)
                    pltpu.sync_copy(rows[b], acc_sp.at[ibuf.at[s, 1]],
                                    add=True)

                    @pl.when(c + NI < CHUNKS)
                    def _refill_idx():
                        pltpu.async_copy(ed_hbm.at[cid, sid, c + NI],
                                         ibuf.at[s], isem.at[s])

                    @pl.when(c + NBUF < CHUNKS)
                    def _next_gather():
                        _drain_i(s2)
                        pltpu.async_copy(g_hbm.at[ibuf.at[s2, 0]], rows[b],
                                         gsem.at[b])
            return _

        lax.fori_loop(0, TT, _visit, None)
        plsc.subcore_barrier()

        def _wb(i, _):
            base = sid * rpt + i * ZCH
            pltpu.sync_copy(acc_sp.at[pl.ds(base, ZCH)],
                            rows[0].at[pl.ds(0, ZCH)])
            pltpu.sync_copy(rows[0].at[pl.ds(0, ZCH)],
                            out_hbm.at[cid, pl.ds(base, ZCH)])
            return _

        lax.fori_loop(0, rpt // ZCH, _wb, None)

    return round_kernel(g, ed_r)


def _tc_mlp(xp, W1, b1r, W2, b2r, degp, N, BM=512):
    NP, D = xp.shape

    def body(x_ref, w1_ref, b1_ref, w2_ref, b2_ref, degp_ref,
             h0_ref, g0_ref, dinv_ref, s_ref):
        i = pl.program_id(0)
        h = lax.dot_general(x_ref[...], w1_ref[...], (((1,), (1,)), ((), ())),
                            precision=lax.Precision.HIGHEST)
        h = jnp.maximum(h + b1_ref[...], 0.0)
        h = lax.dot_general(h, w2_ref[...], (((1,), (1,)), ((), ())),
                            precision=lax.Precision.HIGHEST)
        h = h + b2_ref[...]
        rows = i * BM + lax.broadcasted_iota(jnp.int32, (BM, 1), 0)
        h = jnp.where(rows < N, h, 0.0)
        deg = 1.0 + jnp.sum(degp_ref[...], axis=0, keepdims=True)
        s = lax.rsqrt(deg)
        dinv = 1.0 / deg
        h0_ref[...] = h
        g0_ref[...] = h * s.T
        dinv_ref[...] = dinv.T
        s_ref[...] = s.T

    grid = (NP // BM,)
    return pl.pallas_call(
        body,
        grid=grid,
        in_specs=[
            pl.BlockSpec((BM, D), lambda i: (i, 0)),
            pl.BlockSpec((D, D), lambda i: (0, 0)),
            pl.BlockSpec((1, D), lambda i: (0, 0)),
            pl.BlockSpec((D, D), lambda i: (0, 0)),
            pl.BlockSpec((1, D), lambda i: (0, 0)),
            pl.BlockSpec((NC, BM), lambda i: (0, i)),
        ],
        out_specs=[
            pl.BlockSpec((BM, D), lambda i: (i, 0)),
            pl.BlockSpec((BM, D), lambda i: (i, 0)),
            pl.BlockSpec((BM, 1), lambda i: (i, 0)),
            pl.BlockSpec((BM, 1), lambda i: (i, 0)),
        ],
        out_shape=[
            jax.ShapeDtypeStruct((NP, D), jnp.float32),
            jax.ShapeDtypeStruct((NP, D), jnp.float32),
            jax.ShapeDtypeStruct((NP, 1), jnp.float32),
            jax.ShapeDtypeStruct((NP, 1), jnp.float32),
        ],
    )(xp, W1, b1r, W2, b2r, degp)


def _tc_combine(tmpP, g, dinv, acc, gk, BM=512):
    NP, D = g.shape

    def body(tmp_ref, g_ref, dinv_ref, acc_ref, gk_ref, gn_ref, an_ref):
        tmp = tmp_ref[0] + tmp_ref[1] + g_ref[...]
        gn_ref[...] = dinv_ref[...] * tmp
        an_ref[...] = acc_ref[...] + gk_ref[0, 0] * tmp

    return pl.pallas_call(
        body,
        grid=(NP // BM,),
        in_specs=[
            pl.BlockSpec((NC, BM, D), lambda i: (0, i, 0)),
            pl.BlockSpec((BM, D), lambda i: (i, 0)),
            pl.BlockSpec((BM, 1), lambda i: (i, 0)),
            pl.BlockSpec((BM, D), lambda i: (i, 0)),
            pl.BlockSpec((1, 1), lambda i: (0, 0), memory_space=pltpu.SMEM),
        ],
        out_specs=[
            pl.BlockSpec((BM, D), lambda i: (i, 0)),
            pl.BlockSpec((BM, D), lambda i: (i, 0)),
        ],
        out_shape=[
            jax.ShapeDtypeStruct((NP, D), jnp.float32),
            jax.ShapeDtypeStruct((NP, D), jnp.float32),
        ],
    )(tmpP, g, dinv, acc, gk)


def _tc_final(tmpP, g, acc, s, h0, gk, g0scal, BM=512):
    NP, D = g.shape

    def body(tmp_ref, g_ref, acc_ref, s_ref, h0_ref, gk_ref, g0_ref, out_ref):
        tmp = tmp_ref[0] + tmp_ref[1] + g_ref[...]
        acc = acc_ref[...] + gk_ref[0, 0] * tmp
        out_ref[...] = g0_ref[0, 0] * h0_ref[...] + s_ref[...] * acc

    return pl.pallas_call(
        body,
        grid=(NP // BM,),
        in_specs=[
            pl.BlockSpec((NC, BM, D), lambda i: (0, i, 0)),
            pl.BlockSpec((BM, D), lambda i: (i, 0)),
            pl.BlockSpec((BM, D), lambda i: (i, 0)),
            pl.BlockSpec((BM, 1), lambda i: (i, 0)),
            pl.BlockSpec((BM, D), lambda i: (i, 0)),
            pl.BlockSpec((1, 1), lambda i: (0, 0), memory_space=pltpu.SMEM),
            pl.BlockSpec((1, 1), lambda i: (0, 0), memory_space=pltpu.SMEM),
        ],
        out_specs=pl.BlockSpec((BM, D), lambda i: (i, 0)),
        out_shape=jax.ShapeDtypeStruct((NP, D), jnp.float32),
    )(tmpP, g, acc, s, h0, gk, g0scal)


def kernel(x, edge_index, W1, b1, W2, b2, gamma):
    N, D = x.shape
    E = edge_index.shape[1]
    K = gamma.shape[0] - 1

    NP = _round_up(N + 1, NS * 64)
    EP = _round_up(E, NC * NS * ECHUNK * 2 * NBUF)
    CHUNKS = EP // (NC * NS * ECHUNK)

    src = jnp.concatenate(
        [edge_index[0], jnp.full((EP - E,), N, jnp.int32)]
    ).reshape(NC, NS, CHUNKS, ECHUNK)
    dst = jnp.concatenate(
        [edge_index[1], jnp.full((EP - E,), NP - 1, jnp.int32)]
    ).reshape(NC, NS, CHUNKS, ECHUNK)
    ed = jnp.stack([src, dst], axis=3)
    xp = jnp.pad(x, ((0, NP - N), (0, 0)))

    degp = _sc_degree(ed, NP, CHUNKS)
    h0, g0, dinv, s = _tc_mlp(xp, W1, b1.reshape(1, D), W2,
                              b2.reshape(1, D), degp, N)

    g = g0
    acc = jnp.zeros((NP, D), jnp.float32)
    for k in range(1, K + 1):
        tmpP = _sc_scatter(g, ed, NP, CHUNKS)
        gk = gamma[k].reshape(1, 1)
        if k < K:
            g, acc = _tc_combine(tmpP, g, dinv, acc, gk)
        else:
            out = _tc_final(tmpP, g, acc, s, h0, gk, gamma[0].reshape(1, 1))
    return out[:N]
